# baseline (speedup 1.0000x reference)
import functools

import jax
import jax.numpy as jnp
from jax import lax
from jax.experimental import pallas as pl
from jax.experimental.pallas import tpu as pltpu
from jax.experimental.pallas import tpu_sc as plsc

D = 32
BATCH = 4096
HIST = 200
NC, NS = 2, 16
NW = NC * NS
BAPW = BATCH // NW
NBUF = 2
NGROUP = HIST // NBUF


@functools.partial(
    pl.kernel,
    out_type=jax.ShapeDtypeStruct((HIST, D // 8, BATCH // 128, 8, 128), jnp.float32),
    mesh=plsc.VectorSubcoreMesh(core_axis_name="c", subcore_axis_name="s"),
    scratch_types=[
        pltpu.VMEM((HIST, BAPW), jnp.int32),
        pltpu.VMEM((NBUF, BAPW, D), jnp.float32),
        pltpu.VMEM((NBUF, D // 8, 8, BAPW), jnp.float32),
        [pltpu.SemaphoreType.DMA] * NBUF,
        [pltpu.SemaphoreType.DMA] * NBUF,
    ],
    compiler_params=pltpu.CompilerParams(
        use_tc_tiling_on_sc=False, needs_layout_passes=False
    ),
)
def _gather_kernel(idx_hbm, table_hbm, out_hbm, idx_v, rows_v, trans_v,
                   in_sems, out_sems):
    wid = lax.axis_index("s") * NC + lax.axis_index("c")
    col = pl.multiple_of(wid * BAPW, 8)
    pltpu.sync_copy(idx_hbm.at[:, pl.ds(col, BAPW)], idx_v)

    iota = lax.iota(jnp.int32, 16)
    bcvecs = [iota + (16 * k) for k in range(BAPW // 16)]
    dcols = [jnp.full((16,), d, jnp.int32) for d in range(D)]

    def gather(h, b):
        pltpu.make_async_copy(
            table_hbm.at[idx_v.at[h]], rows_v.at[b], in_sems[b]
        ).start()

    def wait_gather(b):
        pltpu.make_async_copy(
            table_hbm.at[idx_v.at[0]], rows_v.at[b], in_sems[b]
        ).wait()

    def transpose(b):
        rows = rows_v.at[b]
        for d in range(D):
            for k in range(BAPW // 16):
                v = plsc.load_gather(rows, [bcvecs[k], dcols[d]])
                trans_v[b, d // 8, d % 8, pl.ds(16 * k, 16)] = v

    def put(h, b):
        for d8 in range(D // 8):
            pltpu.make_async_copy(
                trans_v.at[b, d8], out_hbm.at[h, d8, wid], out_sems[b]
            ).start()

    def wait_put(b):
        for d8 in range(D // 8):
            pltpu.make_async_copy(
                trans_v.at[b, d8], out_hbm.at[0, d8, wid], out_sems[b]
            ).wait()

    for b in range(NBUF):
        gather(b, b)

    def group(g, carry):
        h0 = g * NBUF
        for b in range(NBUF):
            wait_gather(b)
            transpose(b)
            put(h0 + b, b)
            wait_put(b)
            gather(h0 + NBUF + b, b)
        return carry

    lax.fori_loop(0, NGROUP - 1, group, 0)

    h0 = (NGROUP - 1) * NBUF
    for b in range(NBUF):
        wait_gather(b)
        transpose(b)
        put(h0 + b, b)
        wait_put(b)


def kernel(x, table):
    idx_t = x.astype(jnp.int32).T
    out5 = _gather_kernel(idx_t, table)
    return out5.transpose(2, 4, 0, 1, 3).reshape(BATCH, HIST, D)


# speedup vs baseline: 1.6265x; 1.6265x over previous
import functools

import jax
import jax.numpy as jnp
from jax import lax
from jax.experimental import pallas as pl
from jax.experimental.pallas import tpu as pltpu
from jax.experimental.pallas import tpu_sc as plsc

D = 32
BATCH = 4096
HIST = 200
NC, NS = 2, 16
NW = NC * NS
BAPW = BATCH // NW
NBUF = 2
NGROUP = HIST // NBUF


@functools.partial(
    pl.kernel,
    out_type=jax.ShapeDtypeStruct((HIST, D // 8, BATCH // 128, 8, 128), jnp.float32),
    mesh=plsc.VectorSubcoreMesh(core_axis_name="c", subcore_axis_name="s"),
    scratch_types=[
        pltpu.VMEM((HIST, BAPW), jnp.int32),
        pltpu.VMEM((NBUF, BAPW, D), jnp.float32),
        pltpu.VMEM((NBUF, D // 8, 8, BAPW), jnp.float32),
        [pltpu.SemaphoreType.DMA] * NBUF,
        [pltpu.SemaphoreType.DMA] * NBUF,
    ],
    compiler_params=pltpu.CompilerParams(
        use_tc_tiling_on_sc=False, needs_layout_passes=False
    ),
)
def _gather_kernel(idx_hbm, table_hbm, out_hbm, idx_v, rows_v, trans_v,
                   in_sems, out_sems):
    wid = lax.axis_index("s") * NC + lax.axis_index("c")
    col = pl.multiple_of(wid * BAPW, 8)
    pltpu.sync_copy(idx_hbm.at[:, pl.ds(col, BAPW)], idx_v)

    iota = lax.iota(jnp.int32, 16)

    def gather(h, b):
        pltpu.make_async_copy(
            table_hbm.at[idx_v.at[h]], rows_v.at[b], in_sems[b]
        ).start()

    def wait_gather(b):
        pltpu.make_async_copy(
            table_hbm.at[idx_v.at[0]], rows_v.at[b], in_sems[b]
        ).wait()

    def transpose(b):
        rows = rows_v.at[b]
        trans = trans_v.at[b]

        def tbody(k, carry):
            bcvec = iota + 16 * k
            for d0 in range(D):
                dvec = (iota + d0) & (D - 1)
                v = plsc.load_gather(rows, [bcvec, dvec])
                plsc.store_scatter(trans, [dvec >> 3, dvec & 7, bcvec], v)
            return carry

        lax.fori_loop(0, BAPW // 16, tbody, 0)

    def put(h, b):
        for d8 in range(D // 8):
            pltpu.make_async_copy(
                trans_v.at[b, d8], out_hbm.at[h, d8, wid], out_sems[b]
            ).start()

    def wait_put(b):
        for d8 in range(D // 8):
            pltpu.make_async_copy(
                trans_v.at[b, d8], out_hbm.at[0, d8, wid], out_sems[b]
            ).wait()

    for b in range(NBUF):
        gather(b, b)

    def group(g, carry):
        h0 = g * NBUF
        for b in range(NBUF):
            wait_gather(b)
            transpose(b)
            put(h0 + b, b)
            wait_put(b)
            gather(h0 + NBUF + b, b)
        return carry

    lax.fori_loop(0, NGROUP - 1, group, 0)

    h0 = (NGROUP - 1) * NBUF
    for b in range(NBUF):
        wait_gather(b)
        transpose(b)
        put(h0 + b, b)
        wait_put(b)


def kernel(x, table):
    idx_t = x.astype(jnp.int32).T
    out5 = _gather_kernel(idx_t, table)
    return out5.transpose(2, 4, 0, 1, 3).reshape(BATCH, HIST, D)


# speedup vs baseline: 1.7419x; 1.0709x over previous
import functools

import jax
import jax.numpy as jnp
from jax import lax
from jax.experimental import pallas as pl
from jax.experimental.pallas import tpu as pltpu
from jax.experimental.pallas import tpu_sc as plsc

D = 32
BATCH = 4096
HIST = 200
NC, NS = 2, 16
NW = NC * NS
BAPW = BATCH // NW
NBUF = 2
NGROUP = HIST // NBUF


@functools.partial(
    pl.kernel,
    out_type=jax.ShapeDtypeStruct((HIST, D // 8, BATCH // 128, 8, 128), jnp.float32),
    mesh=plsc.VectorSubcoreMesh(core_axis_name="c", subcore_axis_name="s"),
    scratch_types=[
        pltpu.VMEM((HIST, BAPW), jnp.int32),
        pltpu.VMEM((NBUF, BAPW, D), jnp.float32),
        pltpu.VMEM((NBUF, D // 8, 8, BAPW), jnp.float32),
        [pltpu.SemaphoreType.DMA] * NBUF,
        [pltpu.SemaphoreType.DMA] * NBUF,
    ],
    compiler_params=pltpu.CompilerParams(
        use_tc_tiling_on_sc=False, needs_layout_passes=False
    ),
)
def _gather_kernel(idx_hbm, table_hbm, out_hbm, idx_v, rows_v, trans_v,
                   in_sems, out_sems):
    wid = lax.axis_index("s") * NC + lax.axis_index("c")
    col = pl.multiple_of(wid * BAPW, 8)
    pltpu.sync_copy(idx_hbm.at[:, pl.ds(col, BAPW)], idx_v)

    iota = lax.iota(jnp.int32, 16)

    def gather(h, b):
        pltpu.make_async_copy(
            table_hbm.at[idx_v.at[h]], rows_v.at[b], in_sems[b]
        ).start()

    def wait_gather(b):
        pltpu.make_async_copy(
            table_hbm.at[idx_v.at[0]], rows_v.at[b], in_sems[b]
        ).wait()

    bcvecs = [iota + 16 * k for k in range(BAPW // 16)]

    def transpose(b):
        rows = rows_v.at[b]
        trans = trans_v.at[b]

        def tbody(d0, dvec):
            d8vec = dvec >> 3
            drvec = dvec & 7
            for k in range(BAPW // 16):
                v = plsc.load_gather(rows, [bcvecs[k], dvec])
                plsc.store_scatter(trans, [d8vec, drvec, bcvecs[k]], v)
            return (dvec + 1) & (D - 1)

        lax.fori_loop(0, D, tbody, iota & (D - 1))

    def put(h, b):
        for d8 in range(D // 8):
            pltpu.make_async_copy(
                trans_v.at[b, d8], out_hbm.at[h, d8, wid], out_sems[b]
            ).start()

    def wait_put(b):
        for d8 in range(D // 8):
            pltpu.make_async_copy(
                trans_v.at[b, d8], out_hbm.at[0, d8, wid], out_sems[b]
            ).wait()

    for b in range(NBUF):
        gather(b, b)

    def group(g, carry):
        h0 = g * NBUF
        for b in range(NBUF):
            wait_gather(b)

            @pl.when(g > 0)
            def _():
                wait_put(b)

            transpose(b)
            put(h0 + b, b)
            gather(h0 + NBUF + b, b)
        return carry

    lax.fori_loop(0, NGROUP - 1, group, 0)

    h0 = (NGROUP - 1) * NBUF
    for b in range(NBUF):
        wait_gather(b)
        wait_put(b)
        transpose(b)
        put(h0 + b, b)
    for b in range(NBUF):
        wait_put(b)


def kernel(x, table):
    idx_t = x.astype(jnp.int32).T
    out5 = _gather_kernel(idx_t, table)
    return out5.transpose(2, 4, 0, 1, 3).reshape(BATCH, HIST, D)


# speedup vs baseline: 1.7451x; 1.0018x over previous
import functools

import jax
import jax.numpy as jnp
from jax import lax
from jax.experimental import pallas as pl
from jax.experimental.pallas import tpu as pltpu
from jax.experimental.pallas import tpu_sc as plsc

D = 32
BATCH = 4096
HIST = 200
NC, NS = 2, 16
NW = NC * NS
BAPW = BATCH // NW
NBUF = 2
NGROUP = HIST // NBUF


@functools.partial(
    pl.kernel,
    out_type=jax.ShapeDtypeStruct((HIST, D // 8, BATCH // 128, 8, 128), jnp.float32),
    mesh=plsc.VectorSubcoreMesh(core_axis_name="c", subcore_axis_name="s"),
    scratch_types=[
        pltpu.VMEM((HIST, BAPW), jnp.int32),
        pltpu.VMEM((NBUF, BAPW, D), jnp.float32),
        pltpu.VMEM((NBUF, D // 8, 8, BAPW), jnp.float32),
        [pltpu.SemaphoreType.DMA] * NBUF,
        [pltpu.SemaphoreType.DMA] * NBUF,
    ],
    compiler_params=pltpu.CompilerParams(
        use_tc_tiling_on_sc=False, needs_layout_passes=False
    ),
)
def _gather_kernel(idx_hbm, table_hbm, out_hbm, idx_v, rows_v, trans_v,
                   in_sems, out_sems):
    wid = lax.axis_index("s") * NC + lax.axis_index("c")
    col = pl.multiple_of(wid * BAPW, 8)
    pltpu.sync_copy(idx_hbm.at[:, pl.ds(col, BAPW)], idx_v)

    iota = lax.iota(jnp.int32, 16)

    def gather(h, b):
        pltpu.make_async_copy(
            table_hbm.at[idx_v.at[h]], rows_v.at[b], in_sems[b]
        ).start()

    def wait_gather(b):
        pltpu.make_async_copy(
            table_hbm.at[idx_v.at[0]], rows_v.at[b], in_sems[b]
        ).wait()

    bcvecs = [iota + 16 * k for k in range(BAPW // 16)]

    def transpose(b):
        rows = rows_v.at[b]
        trans = trans_v.at[b]

        def tbody(d0, dvec):
            for _ in range(2):
                d8vec = dvec >> 3
                drvec = dvec & 7
                for k in range(BAPW // 16):
                    v = plsc.load_gather(rows, [bcvecs[k], dvec])
                    plsc.store_scatter(trans, [d8vec, drvec, bcvecs[k]], v)
                dvec = (dvec + 1) & (D - 1)
            return dvec

        lax.fori_loop(0, D // 2, tbody, iota & (D - 1))

    def put(h, b):
        for d8 in range(D // 8):
            pltpu.make_async_copy(
                trans_v.at[b, d8], out_hbm.at[h, d8, wid], out_sems[b]
            ).start()

    def wait_put(b):
        for d8 in range(D // 8):
            pltpu.make_async_copy(
                trans_v.at[b, d8], out_hbm.at[0, d8, wid], out_sems[b]
            ).wait()

    for b in range(NBUF):
        gather(b, b)

    def group(g, carry):
        h0 = g * NBUF
        for b in range(NBUF):
            wait_gather(b)

            @pl.when(g > 0)
            def _():
                wait_put(b)

            transpose(b)
            put(h0 + b, b)
            gather(h0 + NBUF + b, b)
        return carry

    lax.fori_loop(0, NGROUP - 1, group, 0)

    h0 = (NGROUP - 1) * NBUF
    for b in range(NBUF):
        wait_gather(b)
        wait_put(b)
        transpose(b)
        put(h0 + b, b)
    for b in range(NBUF):
        wait_put(b)


def kernel(x, table):
    idx_t = x.astype(jnp.int32).T
    out5 = _gather_kernel(idx_t, table)
    return out5.transpose(2, 4, 0, 1, 3).reshape(BATCH, HIST, D)
